# baseline (speedup 1.0000x reference)
import jax
import jax.numpy as jnp
from jax.experimental import pallas as pl
from jax.experimental.pallas import tpu as pltpu

_BM = 256
_LAG = 8
_NBLK = 32
_RING = _LAG + 1


def _bf(x):
    return x.astype(jnp.bfloat16)


def _critic_kernel(obs_ref, act_ref,
                   w1_hbm, w2_hbm, w3_hbm, wm1a_hbm, wm1e_hbm, wm2_hbm,
                   wm3_hbm,
                   b1_ref, b2_ref, b3_ref, bm1_ref, bm2_ref, bm3_ref,
                   q_ref,
                   w1_v, w2_v, w3_v, wm1a_v, wm1e_v, wm2_v, wm3_v,
                   ring, sem):
    i = pl.program_id(0)
    copies = [
        pltpu.make_async_copy(w1_hbm, w1_v, sem.at[0]),
        pltpu.make_async_copy(w2_hbm, w2_v, sem.at[1]),
        pltpu.make_async_copy(w3_hbm, w3_v, sem.at[2]),
        pltpu.make_async_copy(wm1a_hbm, wm1a_v, sem.at[3]),
        pltpu.make_async_copy(wm1e_hbm, wm1e_v, sem.at[4]),
        pltpu.make_async_copy(wm2_hbm, wm2_v, sem.at[5]),
        pltpu.make_async_copy(wm3_hbm, wm3_v, sem.at[6]),
    ]

    @pl.when(i == 0)
    def _():
        for c in copies:
            c.start()
        copies[0].wait()
        copies[1].wait()
        copies[2].wait()

    @pl.when(i == _LAG)
    def _():
        copies[3].wait()
        copies[4].wait()
        copies[5].wait()
        copies[6].wait()

    @pl.when(i < _NBLK)
    def _():
        h = jnp.dot(_bf(obs_ref[...]), w1_v[...],
                    preferred_element_type=jnp.float32) + b1_ref[...]
        h = jnp.maximum(h, 0.0)
        h = jnp.dot(_bf(h), w2_v[...],
                    preferred_element_type=jnp.float32) + b2_ref[...]
        h = jnp.maximum(h, 0.0)
        emb = jnp.dot(_bf(h), w3_v[...],
                      preferred_element_type=jnp.float32) + b3_ref[...]
        ring[jax.lax.rem(i, _RING)] = _bf(emb)

    @pl.when(i >= _LAG)
    def _():
        j = i - _LAG
        emb = ring[jax.lax.rem(j, _RING)]
        x = (jnp.dot(_bf(act_ref[...]), wm1a_v[...],
                     preferred_element_type=jnp.float32)
             + jnp.dot(emb, wm1e_v[...], preferred_element_type=jnp.float32)
             + bm1_ref[...])
        x = jnp.maximum(x, 0.0)
        x = jnp.dot(_bf(x), wm2_v[...],
                    preferred_element_type=jnp.float32) + bm2_ref[...]
        x = jnp.maximum(x, 0.0)
        q = jnp.dot(_bf(x), wm3_v[...], preferred_element_type=jnp.float32)
        q_ref[...] = q[:, :1] + bm3_ref[...]


def _full_spec(shape):
    nd = len(shape)
    return pl.BlockSpec(shape, lambda i: (0,) * nd)


def _any_spec():
    return pl.BlockSpec(memory_space=pl.ANY)


def kernel(action, obs, W1, b1, W2, b2, W3, b3, Wm1, bm1, Wm2, bm2, Wm3, bm3):
    obs = obs.reshape(-1, W1.shape[0])
    batch = obs.shape[0]
    act = action.reshape(batch, -1)
    a_dim = act.shape[1]
    nblk = batch // _BM
    grid = (nblk + _LAG,)
    params = pltpu.CompilerParams(
        dimension_semantics=("arbitrary",),
        vmem_limit_bytes=62 * 1024 * 1024,
    )
    wm3p = jnp.pad(Wm3, ((0, 0), (0, 256 - Wm3.shape[1])))
    wm1a = Wm1[:a_dim]
    wm1e = Wm1[a_dim:]
    han_out = W3.shape[1]

    q = pl.pallas_call(
        _critic_kernel,
        grid=grid,
        in_specs=[
            pl.BlockSpec((_BM, W1.shape[0]),
                         lambda i: (jnp.minimum(i, _NBLK - 1), 0)),
            pl.BlockSpec((_BM, a_dim),
                         lambda i: (jnp.clip(i - _LAG, 0, _NBLK - 1), 0)),
            _any_spec(), _any_spec(), _any_spec(), _any_spec(), _any_spec(),
            _any_spec(), _any_spec(),
            _full_spec((1, W1.shape[1])),
            _full_spec((1, W2.shape[1])),
            _full_spec((1, W3.shape[1])),
            _full_spec((1, Wm1.shape[1])),
            _full_spec((1, Wm2.shape[1])),
            _full_spec((1, 1)),
        ],
        out_specs=pl.BlockSpec((_BM, 1),
                               lambda i: (jnp.clip(i - _LAG, 0, _NBLK - 1), 0)),
        out_shape=jax.ShapeDtypeStruct((batch, 1), jnp.float32),
        scratch_shapes=[
            pltpu.VMEM(W1.shape, jnp.float32),
            pltpu.VMEM(W2.shape, jnp.float32),
            pltpu.VMEM(W3.shape, jnp.float32),
            pltpu.VMEM(wm1a.shape, jnp.float32),
            pltpu.VMEM(wm1e.shape, jnp.float32),
            pltpu.VMEM(Wm2.shape, jnp.float32),
            pltpu.VMEM(wm3p.shape, jnp.float32),
            pltpu.VMEM((_RING, _BM, han_out), jnp.bfloat16),
            pltpu.SemaphoreType.DMA((7,)),
        ],
        compiler_params=params,
    )(obs, act, W1, W2, W3, wm1a, wm1e, Wm2, wm3p,
      b1.reshape(1, -1), b2.reshape(1, -1), b3.reshape(1, -1),
      bm1.reshape(1, -1), bm2.reshape(1, -1), bm3.reshape(1, -1))
    return q


# speedup vs baseline: 1.0008x; 1.0008x over previous
import jax
import jax.numpy as jnp
from jax.experimental import pallas as pl
from jax.experimental.pallas import tpu as pltpu

_BM = 256
_LAG = 8
_NBLK = 32
_RING = _LAG + 1


def _bf(x):
    return x.astype(jnp.bfloat16)


def _critic_kernel(obs_ref, act_ref,
                   w1_hbm, w2_hbm, w3_hbm, wm1a_hbm, wm1e_hbm, wm2_hbm,
                   wm3_hbm,
                   b1_ref, b2_ref, b3_ref, bm1_ref, bm2_ref, bm3_ref,
                   q_ref,
                   w1_v, w2_v, w3_v, wm1a_v, wm1e_v, wm2_v, wm3_v,
                   ring, sem):
    i = pl.program_id(0)
    copies = [
        pltpu.make_async_copy(w1_hbm, w1_v, sem.at[0]),
        pltpu.make_async_copy(w2_hbm, w2_v, sem.at[1]),
        pltpu.make_async_copy(w3_hbm, w3_v, sem.at[2]),
        pltpu.make_async_copy(wm1a_hbm, wm1a_v, sem.at[3]),
        pltpu.make_async_copy(wm1e_hbm, wm1e_v, sem.at[4]),
        pltpu.make_async_copy(wm2_hbm, wm2_v, sem.at[5]),
        pltpu.make_async_copy(wm3_hbm, wm3_v, sem.at[6]),
    ]

    @pl.when(i == 0)
    def _():
        for c in copies[:3]:
            c.start()
        for c in copies[3:]:
            c.start(priority=1)
        copies[0].wait()
        copies[1].wait()
        copies[2].wait()

    @pl.when(i == _LAG)
    def _():
        copies[3].wait()
        copies[4].wait()
        copies[5].wait()
        copies[6].wait()

    @pl.when(i < _NBLK)
    def _():
        h = jnp.dot(_bf(obs_ref[...]), w1_v[...],
                    preferred_element_type=jnp.float32) + b1_ref[...]
        h = jnp.maximum(h, 0.0)
        h = jnp.dot(_bf(h), w2_v[...],
                    preferred_element_type=jnp.float32) + b2_ref[...]
        h = jnp.maximum(h, 0.0)
        emb = jnp.dot(_bf(h), w3_v[...],
                      preferred_element_type=jnp.float32) + b3_ref[...]
        ring[jax.lax.rem(i, _RING)] = _bf(emb)

    @pl.when(i >= _LAG)
    def _():
        j = i - _LAG
        emb = ring[jax.lax.rem(j, _RING)]
        x = (jnp.dot(_bf(act_ref[...]), wm1a_v[...],
                     preferred_element_type=jnp.float32)
             + jnp.dot(emb, wm1e_v[...], preferred_element_type=jnp.float32)
             + bm1_ref[...])
        x = jnp.maximum(x, 0.0)
        x = jnp.dot(_bf(x), wm2_v[...],
                    preferred_element_type=jnp.float32) + bm2_ref[...]
        x = jnp.maximum(x, 0.0)
        q = jnp.dot(_bf(x), wm3_v[...], preferred_element_type=jnp.float32)
        q_ref[...] = q[:, :1] + bm3_ref[...]


def _full_spec(shape):
    nd = len(shape)
    return pl.BlockSpec(shape, lambda i: (0,) * nd)


def _any_spec():
    return pl.BlockSpec(memory_space=pl.ANY)


def kernel(action, obs, W1, b1, W2, b2, W3, b3, Wm1, bm1, Wm2, bm2, Wm3, bm3):
    obs = obs.reshape(-1, W1.shape[0])
    batch = obs.shape[0]
    act = action.reshape(batch, -1)
    a_dim = act.shape[1]
    nblk = batch // _BM
    grid = (nblk + _LAG,)
    params = pltpu.CompilerParams(
        dimension_semantics=("arbitrary",),
        vmem_limit_bytes=62 * 1024 * 1024,
    )
    wm3p = jnp.pad(Wm3, ((0, 0), (0, 256 - Wm3.shape[1])))
    wm1a = Wm1[:a_dim]
    wm1e = Wm1[a_dim:]
    han_out = W3.shape[1]

    q = pl.pallas_call(
        _critic_kernel,
        grid=grid,
        in_specs=[
            pl.BlockSpec((_BM, W1.shape[0]),
                         lambda i: (jnp.minimum(i, _NBLK - 1), 0)),
            pl.BlockSpec((_BM, a_dim),
                         lambda i: (jnp.clip(i - _LAG, 0, _NBLK - 1), 0)),
            _any_spec(), _any_spec(), _any_spec(), _any_spec(), _any_spec(),
            _any_spec(), _any_spec(),
            _full_spec((1, W1.shape[1])),
            _full_spec((1, W2.shape[1])),
            _full_spec((1, W3.shape[1])),
            _full_spec((1, Wm1.shape[1])),
            _full_spec((1, Wm2.shape[1])),
            _full_spec((1, 1)),
        ],
        out_specs=pl.BlockSpec((_BM, 1),
                               lambda i: (jnp.clip(i - _LAG, 0, _NBLK - 1), 0)),
        out_shape=jax.ShapeDtypeStruct((batch, 1), jnp.float32),
        scratch_shapes=[
            pltpu.VMEM(W1.shape, jnp.float32),
            pltpu.VMEM(W2.shape, jnp.float32),
            pltpu.VMEM(W3.shape, jnp.float32),
            pltpu.VMEM(wm1a.shape, jnp.float32),
            pltpu.VMEM(wm1e.shape, jnp.float32),
            pltpu.VMEM(Wm2.shape, jnp.float32),
            pltpu.VMEM(wm3p.shape, jnp.float32),
            pltpu.VMEM((_RING, _BM, han_out), jnp.bfloat16),
            pltpu.SemaphoreType.DMA((7,)),
        ],
        compiler_params=params,
    )(obs, act, W1, W2, W3, wm1a, wm1e, Wm2, wm3p,
      b1.reshape(1, -1), b2.reshape(1, -1), b3.reshape(1, -1),
      bm1.reshape(1, -1), bm2.reshape(1, -1), bm3.reshape(1, -1))
    return q


# speedup vs baseline: 1.0624x; 1.0616x over previous
import jax
import jax.numpy as jnp
from jax.experimental import pallas as pl
from jax.experimental.pallas import tpu as pltpu

_BM = 512


def _bf(x):
    return x.astype(jnp.bfloat16)


def _critic_kernel(obs_ref, act_ref, w1_ref, b1_ref, w2_ref, b2_ref, w3_ref,
                   b3_ref, wm1a_ref, wm1e_ref, bm1_ref, wm2_ref, bm2_ref,
                   wm3r_ref, bm3_ref, q_ref):
    h = jnp.dot(_bf(obs_ref[...]), w1_ref[...],
                preferred_element_type=jnp.float32) + b1_ref[...]
    h = jnp.maximum(h, 0.0)
    h = jnp.dot(_bf(h), w2_ref[...],
                preferred_element_type=jnp.float32) + b2_ref[...]
    h = jnp.maximum(h, 0.0)
    emb = jnp.dot(_bf(h), w3_ref[...],
                  preferred_element_type=jnp.float32) + b3_ref[...]
    x = (jnp.dot(_bf(act_ref[...]), wm1a_ref[...],
                 preferred_element_type=jnp.float32)
         + jnp.dot(_bf(emb), wm1e_ref[...], preferred_element_type=jnp.float32)
         + bm1_ref[...])
    x = jnp.maximum(x, 0.0)
    x = jnp.dot(_bf(x), wm2_ref[...],
                preferred_element_type=jnp.float32) + bm2_ref[...]
    x = jnp.maximum(x, 0.0)
    q = jnp.sum(x * wm3r_ref[...], axis=1, keepdims=True)
    q_ref[...] = q + bm3_ref[...]


def _row_spec(width):
    return pl.BlockSpec((_BM, width), lambda i: (i, 0))


def _full_spec(shape):
    nd = len(shape)
    return pl.BlockSpec(shape, lambda i: (0,) * nd)


def kernel(action, obs, W1, b1, W2, b2, W3, b3, Wm1, bm1, Wm2, bm2, Wm3, bm3):
    obs = obs.reshape(-1, W1.shape[0])
    batch = obs.shape[0]
    act = action.reshape(batch, -1)
    a_dim = act.shape[1]
    grid = (batch // _BM,)
    params = pltpu.CompilerParams(
        dimension_semantics=("parallel",),
        vmem_limit_bytes=62 * 1024 * 1024,
    )

    q = pl.pallas_call(
        _critic_kernel,
        grid=grid,
        in_specs=[
            _row_spec(W1.shape[0]),
            _row_spec(a_dim),
            _full_spec(W1.shape), _full_spec((1, W1.shape[1])),
            _full_spec(W2.shape), _full_spec((1, W2.shape[1])),
            _full_spec(W3.shape), _full_spec((1, W3.shape[1])),
            _full_spec((a_dim, Wm1.shape[1])),
            _full_spec((Wm1.shape[0] - a_dim, Wm1.shape[1])),
            _full_spec((1, Wm1.shape[1])),
            _full_spec(Wm2.shape), _full_spec((1, Wm2.shape[1])),
            _full_spec((1, Wm2.shape[1])),
            _full_spec((1, 1)),
        ],
        out_specs=_row_spec(1),
        out_shape=jax.ShapeDtypeStruct((batch, 1), jnp.float32),
        compiler_params=params,
    )(obs, act, W1, b1.reshape(1, -1), W2, b2.reshape(1, -1),
      W3, b3.reshape(1, -1), Wm1[:a_dim], Wm1[a_dim:], bm1.reshape(1, -1),
      Wm2, bm2.reshape(1, -1), Wm3.reshape(1, -1), bm3.reshape(1, -1))
    return q


# speedup vs baseline: 1.0852x; 1.0215x over previous
import jax
import jax.numpy as jnp
from jax.experimental import pallas as pl
from jax.experimental.pallas import tpu as pltpu

_BM = 512


def _bf(x):
    return x.astype(jnp.bfloat16)


def _critic_kernel(obs_ref, act_ref, w1_ref, b1_ref, w2_ref, b2_ref, w3_ref,
                   b3_ref, wm1_ref, bm1_ref, wm2_ref, bm2_ref,
                   wm3r_ref, bm3_ref, q_ref):
    a_dim = act_ref.shape[1]
    h = jnp.dot(_bf(obs_ref[...]), w1_ref[...],
                preferred_element_type=jnp.float32) + b1_ref[...]
    h = jnp.maximum(h, 0.0)
    h = jnp.dot(_bf(h), w2_ref[...],
                preferred_element_type=jnp.float32) + b2_ref[...]
    h = jnp.maximum(h, 0.0)
    emb = jnp.dot(_bf(h), w3_ref[...],
                  preferred_element_type=jnp.float32) + b3_ref[...]
    x = (jnp.dot(_bf(act_ref[...]), wm1_ref[:a_dim, :],
                 preferred_element_type=jnp.float32)
         + jnp.dot(_bf(emb), wm1_ref[a_dim:, :],
                   preferred_element_type=jnp.float32)
         + bm1_ref[...])
    x = jnp.maximum(x, 0.0)
    x = jnp.dot(_bf(x), wm2_ref[...],
                preferred_element_type=jnp.float32) + bm2_ref[...]
    x = jnp.maximum(x, 0.0)
    q = jnp.sum(x * wm3r_ref[...], axis=1, keepdims=True)
    q_ref[...] = q + bm3_ref[...]


def _row_spec(width):
    return pl.BlockSpec((_BM, width), lambda i: (i, 0))


def _full_spec(shape):
    nd = len(shape)
    return pl.BlockSpec(shape, lambda i: (0,) * nd)


def kernel(action, obs, W1, b1, W2, b2, W3, b3, Wm1, bm1, Wm2, bm2, Wm3, bm3):
    obs = obs.reshape(-1, W1.shape[0])
    batch = obs.shape[0]
    act = action.reshape(batch, -1)
    a_dim = act.shape[1]
    grid = (batch // _BM,)
    params = pltpu.CompilerParams(
        dimension_semantics=("parallel",),
        vmem_limit_bytes=62 * 1024 * 1024,
    )

    q = pl.pallas_call(
        _critic_kernel,
        grid=grid,
        in_specs=[
            _row_spec(W1.shape[0]),
            _row_spec(a_dim),
            _full_spec(W1.shape), _full_spec((1, W1.shape[1])),
            _full_spec(W2.shape), _full_spec((1, W2.shape[1])),
            _full_spec(W3.shape), _full_spec((1, W3.shape[1])),
            _full_spec(Wm1.shape),
            _full_spec((1, Wm1.shape[1])),
            _full_spec(Wm2.shape), _full_spec((1, Wm2.shape[1])),
            _full_spec((1, Wm2.shape[1])),
            _full_spec((1, 1)),
        ],
        out_specs=_row_spec(1),
        out_shape=jax.ShapeDtypeStruct((batch, 1), jnp.float32),
        compiler_params=params,
    )(obs, act, W1, b1.reshape(1, -1), W2, b2.reshape(1, -1),
      W3, b3.reshape(1, -1), Wm1, bm1.reshape(1, -1),
      Wm2, bm2.reshape(1, -1), Wm3.reshape(1, -1), bm3.reshape(1, -1))
    return q


# speedup vs baseline: 1.0941x; 1.0081x over previous
import jax
import jax.numpy as jnp
from jax.experimental import pallas as pl
from jax.experimental.pallas import tpu as pltpu

_BM = 512
_LAG = 3
_NBLK = 16
_RING = _LAG + 1


def _bf(x):
    return x.astype(jnp.bfloat16)


def _critic_kernel(obs_ref, act_ref,
                   w1_hbm, w2_hbm, w3_hbm, wm1_hbm, wm2_hbm,
                   b1_ref, b2_ref, b3_ref, bm1_ref, bm2_ref,
                   wm3r_ref, bm3_ref,
                   q_ref,
                   w1_v, w2_v, w3_v, wm1_v, wm2_v, ring, sem):
    i = pl.program_id(0)
    a_dim = act_ref.shape[1]
    copies = [
        pltpu.make_async_copy(w1_hbm, w1_v, sem.at[0]),
        pltpu.make_async_copy(w2_hbm, w2_v, sem.at[1]),
        pltpu.make_async_copy(w3_hbm, w3_v, sem.at[2]),
        pltpu.make_async_copy(wm1_hbm, wm1_v, sem.at[3]),
        pltpu.make_async_copy(wm2_hbm, wm2_v, sem.at[4]),
    ]

    @pl.when(i == 0)
    def _():
        for c in copies:
            c.start()
        copies[0].wait()
        copies[1].wait()
        copies[2].wait()

    @pl.when(i == _LAG)
    def _():
        copies[3].wait()
        copies[4].wait()

    @pl.when(i < _NBLK)
    def _():
        h = jnp.dot(_bf(obs_ref[...]), w1_v[...],
                    preferred_element_type=jnp.float32) + b1_ref[...]
        h = jnp.maximum(h, 0.0)
        h = jnp.dot(_bf(h), w2_v[...],
                    preferred_element_type=jnp.float32) + b2_ref[...]
        h = jnp.maximum(h, 0.0)
        emb = jnp.dot(_bf(h), w3_v[...],
                      preferred_element_type=jnp.float32) + b3_ref[...]
        ring[jax.lax.rem(i, _RING)] = _bf(emb)

    @pl.when(i >= _LAG)
    def _():
        j = i - _LAG
        emb = ring[jax.lax.rem(j, _RING)]
        x = (jnp.dot(_bf(act_ref[...]), wm1_v[:a_dim, :],
                     preferred_element_type=jnp.float32)
             + jnp.dot(emb, wm1_v[a_dim:, :],
                       preferred_element_type=jnp.float32)
             + bm1_ref[...])
        x = jnp.maximum(x, 0.0)
        x = jnp.dot(_bf(x), wm2_v[...],
                    preferred_element_type=jnp.float32) + bm2_ref[...]
        x = jnp.maximum(x, 0.0)
        q = jnp.sum(x * wm3r_ref[...], axis=1, keepdims=True)
        q_ref[...] = q + bm3_ref[...]


def _full_spec(shape):
    nd = len(shape)
    return pl.BlockSpec(shape, lambda i: (0,) * nd)


def _any_spec():
    return pl.BlockSpec(memory_space=pl.ANY)


def kernel(action, obs, W1, b1, W2, b2, W3, b3, Wm1, bm1, Wm2, bm2, Wm3, bm3):
    obs = obs.reshape(-1, W1.shape[0])
    batch = obs.shape[0]
    act = action.reshape(batch, -1)
    a_dim = act.shape[1]
    nblk = batch // _BM
    grid = (nblk + _LAG,)
    params = pltpu.CompilerParams(
        dimension_semantics=("arbitrary",),
        vmem_limit_bytes=62 * 1024 * 1024,
    )
    han_out = W3.shape[1]

    q = pl.pallas_call(
        _critic_kernel,
        grid=grid,
        in_specs=[
            pl.BlockSpec((_BM, W1.shape[0]),
                         lambda i: (jnp.minimum(i, _NBLK - 1), 0)),
            pl.BlockSpec((_BM, a_dim),
                         lambda i: (jnp.clip(i - _LAG, 0, _NBLK - 1), 0)),
            _any_spec(), _any_spec(), _any_spec(), _any_spec(), _any_spec(),
            _full_spec((1, W1.shape[1])),
            _full_spec((1, W2.shape[1])),
            _full_spec((1, W3.shape[1])),
            _full_spec((1, Wm1.shape[1])),
            _full_spec((1, Wm2.shape[1])),
            _full_spec((1, Wm2.shape[1])),
            _full_spec((1, 1)),
        ],
        out_specs=pl.BlockSpec((_BM, 1),
                               lambda i: (jnp.clip(i - _LAG, 0, _NBLK - 1), 0)),
        out_shape=jax.ShapeDtypeStruct((batch, 1), jnp.float32),
        scratch_shapes=[
            pltpu.VMEM(W1.shape, jnp.float32),
            pltpu.VMEM(W2.shape, jnp.float32),
            pltpu.VMEM(W3.shape, jnp.float32),
            pltpu.VMEM(Wm1.shape, jnp.float32),
            pltpu.VMEM(Wm2.shape, jnp.float32),
            pltpu.VMEM((_RING, _BM, han_out), jnp.bfloat16),
            pltpu.SemaphoreType.DMA((5,)),
        ],
        compiler_params=params,
    )(obs, act, W1, W2, W3, Wm1, Wm2,
      b1.reshape(1, -1), b2.reshape(1, -1), b3.reshape(1, -1),
      bm1.reshape(1, -1), bm2.reshape(1, -1),
      Wm3.reshape(1, -1), bm3.reshape(1, -1))
    return q
